# baseline (speedup 1.0000x reference)
import jax
import jax.numpy as jnp
from jax.experimental import pallas as pl
from jax.experimental.pallas import tpu as pltpu

_B = 16
_MAXLEN = 4096
_T = 32768
_IN = 4
_HID = 64
_D = 16
_BLK = 4096
_NBLK = _T // _BLK


def _spcov_body(bidx_ref, feats_ref, W1t_ref, b1_ref, W2t_ref, b2_ref,
                out_ref, lfeat_ref, z_scr, offs_ref, cnts_ref):
    i = pl.program_id(0)

    @pl.when(i == 0)
    def _prep():
        bidx = bidx_ref[...]
        prev = jnp.int32(0)
        for b in range(_B):
            nxt = (jnp.sum((bidx <= b).astype(jnp.int32)) if b < _B - 1
                   else jnp.int32(_T))
            off = prev if b > 0 else jnp.int32(0)
            offs_ref[0, b] = off // 128
            offs_ref[1, b] = off % 128
            cnts_ref[0, b] = nxt - off
            prev = nxt
        z_scr[:, pl.ds(_T, _MAXLEN + 128)] = jnp.zeros(
            (_D, _MAXLEN + 128), jnp.float32)

    @pl.when(i < _NBLK)
    def _mlp():
        fs = feats_ref[...]
        h = jnp.maximum(
            jnp.dot(W1t_ref[...], fs, preferred_element_type=jnp.float32)
            + b1_ref[...], 0.0)
        z = (jnp.dot(W2t_ref[...], h, preferred_element_type=jnp.float32)
             + b2_ref[...])
        z_scr[:, pl.ds(i * _BLK, _BLK)] = z

    @pl.when(i >= _NBLK)
    def _cov():
        b = i - _NBLK
        phase = offs_ref[1, b]
        cnt = cnts_ref[0, b]
        zw = z_scr[:, pl.ds(offs_ref[0, b] * 128, _MAXLEN + 128)]
        n = jax.lax.broadcasted_iota(jnp.int32, (1, _MAXLEN + 128), 1)
        maskf = ((n >= phase) & (n < phase + cnt)).astype(jnp.float32)
        zm = zw * maskf
        lfeat_ref[...] = pltpu.roll(
            zm, (_MAXLEN + 128) - phase, axis=1)[:, :_MAXLEN][None]
        cf = cnt.astype(jnp.float32)
        mean = jnp.sum(zm, axis=1, keepdims=True) / jnp.maximum(cf, 1.0)
        c = (zw - mean) * maskf
        cov = jax.lax.dot_general(
            c, c, (((1,), (1,)), ((), ())),
            preferred_element_type=jnp.float32) / jnp.maximum(cf - 1.0, 1.0)
        v = jnp.sign(cov) * jnp.sqrt(jnp.abs(cov) + 1e-12)
        out_ref[...] = (v / jnp.maximum(jnp.sqrt(jnp.sum(v * v)), 1e-12))[None]


def kernel(feats, batch_idx, W1, b1, W2, b2):
    bidx2d = batch_idx.reshape(_T // 128, 128)
    out, lfeat = pl.pallas_call(
        _spcov_body,
        grid=(_NBLK + _B,),
        in_specs=[
            pl.BlockSpec((_T // 128, 128), lambda i: (0, 0)),
            pl.BlockSpec((_IN, _BLK), lambda i: (0, jnp.minimum(i, _NBLK - 1))),
            pl.BlockSpec((_HID, _IN), lambda i: (0, 0)),
            pl.BlockSpec((_HID, 1), lambda i: (0, 0)),
            pl.BlockSpec((_D, _HID), lambda i: (0, 0)),
            pl.BlockSpec((_D, 1), lambda i: (0, 0)),
        ],
        out_specs=[
            pl.BlockSpec((1, _D, _D), lambda i: (jnp.maximum(i - _NBLK, 0), 0, 0)),
            pl.BlockSpec((1, _D, _MAXLEN),
                         lambda i: (jnp.maximum(i - _NBLK, 0), 0, 0)),
        ],
        out_shape=[
            jax.ShapeDtypeStruct((_B, _D, _D), jnp.float32),
            jax.ShapeDtypeStruct((_B, _D, _MAXLEN), jnp.float32),
        ],
        scratch_shapes=[
            pltpu.VMEM((_D, _T + _MAXLEN + 128), jnp.float32),
            pltpu.SMEM((2, _B), jnp.int32),
            pltpu.SMEM((1, _B), jnp.int32),
        ],
        compiler_params=pltpu.CompilerParams(
            dimension_semantics=("arbitrary",)),
    )(bidx2d, jnp.zeros((_IN, _T), jnp.float32) + feats[0, 0], W1.T, b1.reshape(_HID, 1), W2.T, b2.reshape(_D, 1))
    return out.reshape(_B, _D * _D), lfeat.transpose(0, 2, 1)


# speedup vs baseline: 1.2276x; 1.2276x over previous
import jax
import jax.numpy as jnp
from jax.experimental import pallas as pl
from jax.experimental.pallas import tpu as pltpu

_B = 16
_MAXLEN = 4096
_T = 32768
_IN = 4
_HID = 64
_D = 16
_BLK = 8192
_NBLK = _T // _BLK


def _spcov_body(bidx_ref, feats_ref, W1t_ref, b1_ref, W2t_ref, b2_ref,
                out_ref, lfeat_ref, z_scr, offs_ref, cnts_ref):
    i = pl.program_id(0)

    @pl.when(i == 0)
    def _prep():
        bidx = bidx_ref[...]
        prev = jnp.int32(0)
        for b in range(_B):
            nxt = (jnp.sum((bidx <= b).astype(jnp.int32)) if b < _B - 1
                   else jnp.int32(_T))
            off = prev if b > 0 else jnp.int32(0)
            offs_ref[0, b] = off // 128
            offs_ref[1, b] = off % 128
            cnts_ref[0, b] = nxt - off
            prev = nxt
        z_scr[:, pl.ds(_T, _MAXLEN + 128)] = jnp.zeros(
            (_D, _MAXLEN + 128), jnp.float32)

    @pl.when(i < _NBLK)
    def _mlp():
        fs = feats_ref[...]
        h = jnp.maximum(
            jnp.dot(W1t_ref[...], fs, preferred_element_type=jnp.float32)
            + b1_ref[...], 0.0)
        z = (jnp.dot(W2t_ref[...], h, preferred_element_type=jnp.float32)
             + b2_ref[...])
        z_scr[:, pl.ds(i * _BLK, _BLK)] = z

    @pl.when(i >= _NBLK)
    def _cov():
        b = i - _NBLK
        phase = offs_ref[1, b]
        cnt = cnts_ref[0, b]
        zw = z_scr[:, pl.ds(offs_ref[0, b] * 128, _MAXLEN + 128)]
        n = jax.lax.broadcasted_iota(jnp.int32, (1, _MAXLEN + 128), 1)
        maskf = ((n >= phase) & (n < phase + cnt)).astype(jnp.float32)
        zm = zw * maskf
        lfeat_ref[...] = pltpu.roll(
            zm, (_MAXLEN + 128) - phase, axis=1)[:, :_MAXLEN][None]
        cf = cnt.astype(jnp.float32)
        mean = jnp.sum(zm, axis=1, keepdims=True) / jnp.maximum(cf, 1.0)
        c = (zw - mean) * maskf
        cov = jax.lax.dot_general(
            c, c, (((1,), (1,)), ((), ())),
            preferred_element_type=jnp.float32) / jnp.maximum(cf - 1.0, 1.0)
        v = jnp.sign(cov) * jnp.sqrt(jnp.abs(cov) + 1e-12)
        out_ref[...] = (v / jnp.maximum(jnp.sqrt(jnp.sum(v * v)), 1e-12))[None]


def kernel(feats, batch_idx, W1, b1, W2, b2):
    bidx2d = batch_idx.reshape(_T // 128, 128)
    out, lfeat = pl.pallas_call(
        _spcov_body,
        grid=(_NBLK + _B,),
        in_specs=[
            pl.BlockSpec((_T // 128, 128), lambda i: (0, 0)),
            pl.BlockSpec((_IN, _BLK), lambda i: (0, jnp.minimum(i, _NBLK - 1))),
            pl.BlockSpec((_HID, _IN), lambda i: (0, 0)),
            pl.BlockSpec((_HID, 1), lambda i: (0, 0)),
            pl.BlockSpec((_D, _HID), lambda i: (0, 0)),
            pl.BlockSpec((_D, 1), lambda i: (0, 0)),
        ],
        out_specs=[
            pl.BlockSpec((1, _D, _D), lambda i: (jnp.maximum(i - _NBLK, 0), 0, 0)),
            pl.BlockSpec((1, _D, _MAXLEN),
                         lambda i: (jnp.maximum(i - _NBLK, 0), 0, 0)),
        ],
        out_shape=[
            jax.ShapeDtypeStruct((_B, _D, _D), jnp.float32),
            jax.ShapeDtypeStruct((_B, _D, _MAXLEN), jnp.float32),
        ],
        scratch_shapes=[
            pltpu.VMEM((_D, _T + _MAXLEN + 128), jnp.float32),
            pltpu.SMEM((2, _B), jnp.int32),
            pltpu.SMEM((1, _B), jnp.int32),
        ],
        compiler_params=pltpu.CompilerParams(
            dimension_semantics=("arbitrary",)),
    )(bidx2d, feats.T, W1.T, b1.reshape(_HID, 1), W2.T, b2.reshape(_D, 1))
    return out.reshape(_B, _D * _D), lfeat.transpose(0, 2, 1)


# speedup vs baseline: 1.2721x; 1.0362x over previous
import jax
import jax.numpy as jnp
from jax.experimental import pallas as pl
from jax.experimental.pallas import tpu as pltpu

_B = 16
_MAXLEN = 4096
_T = 32768
_IN = 4
_HID = 64
_D = 16
_BLK = 32768
_NBLK = _T // _BLK


def _spcov_body(bidx_ref, feats_ref, W1t_ref, b1_ref, W2t_ref, b2_ref,
                out_ref, lfeat_ref, z_scr, offs_ref, cnts_ref):
    i = pl.program_id(0)

    @pl.when(i == 0)
    def _prep():
        bidx = bidx_ref[...]
        prev = jnp.int32(0)
        for b in range(_B):
            nxt = (jnp.sum((bidx <= b).astype(jnp.int32)) if b < _B - 1
                   else jnp.int32(_T))
            off = prev if b > 0 else jnp.int32(0)
            offs_ref[0, b] = off // 128
            offs_ref[1, b] = off % 128
            cnts_ref[0, b] = nxt - off
            prev = nxt
        z_scr[:, pl.ds(_T, _MAXLEN + 128)] = jnp.zeros(
            (_D, _MAXLEN + 128), jnp.float32)

    @pl.when(i < _NBLK)
    def _mlp():
        fs = feats_ref[...]
        h = jnp.maximum(
            jnp.dot(W1t_ref[...], fs, preferred_element_type=jnp.float32)
            + b1_ref[...], 0.0)
        z = (jnp.dot(W2t_ref[...], h, preferred_element_type=jnp.float32)
             + b2_ref[...])
        z_scr[:, pl.ds(i * _BLK, _BLK)] = z

    @pl.when(i >= _NBLK)
    def _cov():
        b = i - _NBLK
        phase = offs_ref[1, b]
        cnt = cnts_ref[0, b]
        zw = z_scr[:, pl.ds(offs_ref[0, b] * 128, _MAXLEN + 128)]
        n = jax.lax.broadcasted_iota(jnp.int32, (1, _MAXLEN + 128), 1)
        maskf = ((n >= phase) & (n < phase + cnt)).astype(jnp.float32)
        zm = zw * maskf
        lfeat_ref[...] = pltpu.roll(
            zm, (_MAXLEN + 128) - phase, axis=1)[:, :_MAXLEN][None]
        cf = cnt.astype(jnp.float32)
        mean = jnp.sum(zm, axis=1, keepdims=True) / jnp.maximum(cf, 1.0)
        c = (zw - mean) * maskf
        cov = jax.lax.dot_general(
            c, c, (((1,), (1,)), ((), ())),
            preferred_element_type=jnp.float32) / jnp.maximum(cf - 1.0, 1.0)
        v = jnp.sign(cov) * jnp.sqrt(jnp.abs(cov) + 1e-12)
        out_ref[...] = (v / jnp.maximum(jnp.sqrt(jnp.sum(v * v)), 1e-12))[None]


def kernel(feats, batch_idx, W1, b1, W2, b2):
    bidx2d = batch_idx.reshape(_T // 128, 128)
    out, lfeat = pl.pallas_call(
        _spcov_body,
        grid=(_NBLK + _B,),
        in_specs=[
            pl.BlockSpec((_T // 128, 128), lambda i: (0, 0)),
            pl.BlockSpec((_IN, _BLK), lambda i: (0, jnp.minimum(i, _NBLK - 1))),
            pl.BlockSpec((_HID, _IN), lambda i: (0, 0)),
            pl.BlockSpec((_HID, 1), lambda i: (0, 0)),
            pl.BlockSpec((_D, _HID), lambda i: (0, 0)),
            pl.BlockSpec((_D, 1), lambda i: (0, 0)),
        ],
        out_specs=[
            pl.BlockSpec((1, _D, _D), lambda i: (jnp.maximum(i - _NBLK, 0), 0, 0)),
            pl.BlockSpec((1, _D, _MAXLEN),
                         lambda i: (jnp.maximum(i - _NBLK, 0), 0, 0)),
        ],
        out_shape=[
            jax.ShapeDtypeStruct((_B, _D, _D), jnp.float32),
            jax.ShapeDtypeStruct((_B, _D, _MAXLEN), jnp.float32),
        ],
        scratch_shapes=[
            pltpu.VMEM((_D, _T + _MAXLEN + 128), jnp.float32),
            pltpu.SMEM((2, _B), jnp.int32),
            pltpu.SMEM((1, _B), jnp.int32),
        ],
        compiler_params=pltpu.CompilerParams(
            dimension_semantics=("arbitrary",)),
    )(bidx2d, feats.T, W1.T, b1.reshape(_HID, 1), W2.T, b2.reshape(_D, 1))
    return out.reshape(_B, _D * _D), lfeat.transpose(0, 2, 1)


# speedup vs baseline: 1.5753x; 1.2383x over previous
import jax
import jax.numpy as jnp
from jax.experimental import pallas as pl
from jax.experimental.pallas import tpu as pltpu

_B = 16
_MAXLEN = 4096
_T = 32768
_IN = 4
_HID = 64
_D = 16
_BLK = 32768
_NBLK = _T // _BLK
_CB = 2


def _spcov_body(bidx_ref, feats_ref, W1t_ref, b1_ref, W2t_ref, b2_ref,
                out_ref, lfeat_ref, z_scr, offs_ref, cnts_ref):
    i = pl.program_id(0)

    @pl.when(i == 0)
    def _prep():
        bidx = bidx_ref[...]
        prev = jnp.int32(0)
        for b in range(_B):
            nxt = (jnp.sum((bidx <= b).astype(jnp.int32)) if b < _B - 1
                   else jnp.int32(_T))
            off = prev if b > 0 else jnp.int32(0)
            offs_ref[0, b] = off // 128
            offs_ref[1, b] = off % 128
            cnts_ref[0, b] = nxt - off
            prev = nxt
        z_scr[:, pl.ds(_T, _MAXLEN + 128)] = jnp.zeros(
            (_D, _MAXLEN + 128), jnp.float32)

    @pl.when(i < _NBLK)
    def _mlp():
        fs = feats_ref[...]
        h = jnp.maximum(
            jnp.dot(W1t_ref[...], fs, preferred_element_type=jnp.float32)
            + b1_ref[...], 0.0)
        z = (jnp.dot(W2t_ref[...], h, preferred_element_type=jnp.float32)
             + b2_ref[...])
        z_scr[:, pl.ds(i * _BLK, _BLK)] = z

    @pl.when(i >= _NBLK)
    def _cov():
        for bb in range(_CB):
            b = (i - _NBLK) * _CB + bb
            phase = offs_ref[1, b]
            cnt = cnts_ref[0, b]
            zw = z_scr[:, pl.ds(offs_ref[0, b] * 128, _MAXLEN + 128)]
            n = jax.lax.broadcasted_iota(jnp.int32, (1, _MAXLEN + 128), 1)
            maskf = ((n >= phase) & (n < phase + cnt)).astype(jnp.float32)
            zm = zw * maskf
            lfeat_ref[bb] = pltpu.roll(
                zm, (_MAXLEN + 128) - phase, axis=1)[:, :_MAXLEN]
            cf = cnt.astype(jnp.float32)
            mean = jnp.sum(zm, axis=1, keepdims=True) / jnp.maximum(cf, 1.0)
            c = (zw - mean) * maskf
            cov = jax.lax.dot_general(
                c, c, (((1,), (1,)), ((), ())),
                preferred_element_type=jnp.float32) / jnp.maximum(cf - 1.0, 1.0)
            v = jnp.sign(cov) * jnp.sqrt(jnp.abs(cov) + 1e-12)
            out_ref[bb] = v / jnp.maximum(jnp.sqrt(jnp.sum(v * v)), 1e-12)


def kernel(feats, batch_idx, W1, b1, W2, b2):
    bidx2d = batch_idx.reshape(_T // 128, 128)
    out, lfeat = pl.pallas_call(
        _spcov_body,
        grid=(_NBLK + _B // _CB,),
        in_specs=[
            pl.BlockSpec((_T // 128, 128), lambda i: (0, 0)),
            pl.BlockSpec((_IN, _BLK), lambda i: (0, jnp.minimum(i, _NBLK - 1))),
            pl.BlockSpec((_HID, _IN), lambda i: (0, 0)),
            pl.BlockSpec((_HID, 1), lambda i: (0, 0)),
            pl.BlockSpec((_D, _HID), lambda i: (0, 0)),
            pl.BlockSpec((_D, 1), lambda i: (0, 0)),
        ],
        out_specs=[
            pl.BlockSpec((_CB, _D, _D),
                         lambda i: (jnp.maximum(i - _NBLK, 0), 0, 0)),
            pl.BlockSpec((_CB, _D, _MAXLEN),
                         lambda i: (jnp.maximum(i - _NBLK, 0), 0, 0)),
        ],
        out_shape=[
            jax.ShapeDtypeStruct((_B, _D, _D), jnp.float32),
            jax.ShapeDtypeStruct((_B, _D, _MAXLEN), jnp.float32),
        ],
        scratch_shapes=[
            pltpu.VMEM((_D, _T + _MAXLEN + 128), jnp.float32),
            pltpu.SMEM((2, _B), jnp.int32),
            pltpu.SMEM((1, _B), jnp.int32),
        ],
        compiler_params=pltpu.CompilerParams(
            dimension_semantics=("arbitrary",)),
    )(bidx2d, feats.T, W1.T, b1.reshape(_HID, 1), W2.T, b2.reshape(_D, 1))
    return out.reshape(_B, _D * _D), lfeat.transpose(0, 2, 1)


# speedup vs baseline: 1.7795x; 1.1296x over previous
import jax
import jax.numpy as jnp
from jax.experimental import pallas as pl
from jax.experimental.pallas import tpu as pltpu

_B = 16
_MAXLEN = 4096
_T = 32768
_IN = 4
_HID = 64
_D = 16
_BLK = 32768
_NBLK = _T // _BLK
_CB = 4


def _spcov_body(bidx_ref, feats_ref, W1t_ref, b1_ref, W2t_ref, b2_ref,
                out_ref, lfeat_ref, z_scr, offs_ref, cnts_ref):
    i = pl.program_id(0)

    @pl.when(i == 0)
    def _prep():
        bidx = bidx_ref[...]
        prev = jnp.int32(0)
        for b in range(_B):
            nxt = (jnp.sum((bidx <= b).astype(jnp.int32)) if b < _B - 1
                   else jnp.int32(_T))
            off = prev if b > 0 else jnp.int32(0)
            offs_ref[0, b] = off // 128
            offs_ref[1, b] = off % 128
            cnts_ref[0, b] = nxt - off
            prev = nxt
        z_scr[:, pl.ds(_T, _MAXLEN + 128)] = jnp.zeros(
            (_D, _MAXLEN + 128), jnp.float32)

    @pl.when(i < _NBLK)
    def _mlp():
        fs = feats_ref[...]
        h = jnp.maximum(
            jnp.dot(W1t_ref[...], fs, preferred_element_type=jnp.float32)
            + b1_ref[...], 0.0)
        z = (jnp.dot(W2t_ref[...], h, preferred_element_type=jnp.float32)
             + b2_ref[...])
        z_scr[:, pl.ds(i * _BLK, _BLK)] = z

    @pl.when(i >= _NBLK)
    def _cov():
        for bb in range(_CB):
            b = (i - _NBLK) * _CB + bb
            phase = offs_ref[1, b]
            cnt = cnts_ref[0, b]
            zw = z_scr[:, pl.ds(offs_ref[0, b] * 128, _MAXLEN + 128)]
            n = jax.lax.broadcasted_iota(jnp.int32, (1, _MAXLEN + 128), 1)
            maskf = ((n >= phase) & (n < phase + cnt)).astype(jnp.float32)
            zm = zw * maskf
            lfeat_ref[bb] = pltpu.roll(
                zm, (_MAXLEN + 128) - phase, axis=1)[:, :_MAXLEN]
            cf = cnt.astype(jnp.float32)
            mean = jnp.sum(zm, axis=1, keepdims=True) / jnp.maximum(cf, 1.0)
            c = (zw - mean) * maskf
            cov = jax.lax.dot_general(
                c, c, (((1,), (1,)), ((), ())),
                preferred_element_type=jnp.float32) / jnp.maximum(cf - 1.0, 1.0)
            v = jnp.sign(cov) * jnp.sqrt(jnp.abs(cov) + 1e-12)
            out_ref[bb] = v / jnp.maximum(jnp.sqrt(jnp.sum(v * v)), 1e-12)


def kernel(feats, batch_idx, W1, b1, W2, b2):
    bidx2d = batch_idx.reshape(_T // 128, 128)
    out, lfeat = pl.pallas_call(
        _spcov_body,
        grid=(_NBLK + _B // _CB,),
        in_specs=[
            pl.BlockSpec((_T // 128, 128), lambda i: (0, 0)),
            pl.BlockSpec((_IN, _BLK), lambda i: (0, jnp.minimum(i, _NBLK - 1))),
            pl.BlockSpec((_HID, _IN), lambda i: (0, 0)),
            pl.BlockSpec((_HID, 1), lambda i: (0, 0)),
            pl.BlockSpec((_D, _HID), lambda i: (0, 0)),
            pl.BlockSpec((_D, 1), lambda i: (0, 0)),
        ],
        out_specs=[
            pl.BlockSpec((_CB, _D, _D),
                         lambda i: (jnp.maximum(i - _NBLK, 0), 0, 0)),
            pl.BlockSpec((_CB, _D, _MAXLEN),
                         lambda i: (jnp.maximum(i - _NBLK, 0), 0, 0)),
        ],
        out_shape=[
            jax.ShapeDtypeStruct((_B, _D, _D), jnp.float32),
            jax.ShapeDtypeStruct((_B, _D, _MAXLEN), jnp.float32),
        ],
        scratch_shapes=[
            pltpu.VMEM((_D, _T + _MAXLEN + 128), jnp.float32),
            pltpu.SMEM((2, _B), jnp.int32),
            pltpu.SMEM((1, _B), jnp.int32),
        ],
        compiler_params=pltpu.CompilerParams(
            dimension_semantics=("arbitrary",)),
    )(bidx2d, feats.T, W1.T, b1.reshape(_HID, 1), W2.T, b2.reshape(_D, 1))
    return out.reshape(_B, _D * _D), lfeat.transpose(0, 2, 1)


# speedup vs baseline: 1.7857x; 1.0035x over previous
import jax
import jax.numpy as jnp
from jax.experimental import pallas as pl
from jax.experimental.pallas import tpu as pltpu

_B = 16
_MAXLEN = 4096
_T = 32768
_IN = 4
_HID = 64
_D = 16
_BLK = 32768
_NBLK = _T // _BLK
_CB = 8


def _spcov_body(bidx_ref, feats_ref, W1t_ref, b1_ref, W2t_ref, b2_ref,
                out_ref, lfeat_ref, z_scr, offs_ref, cnts_ref):
    i = pl.program_id(0)

    @pl.when(i == 0)
    def _prep():
        bidx = bidx_ref[...]
        prev = jnp.int32(0)
        for b in range(_B):
            nxt = (jnp.sum((bidx <= b).astype(jnp.int32)) if b < _B - 1
                   else jnp.int32(_T))
            off = prev if b > 0 else jnp.int32(0)
            offs_ref[0, b] = off // 128
            offs_ref[1, b] = off % 128
            cnts_ref[0, b] = nxt - off
            prev = nxt
        z_scr[:, pl.ds(_T, _MAXLEN + 128)] = jnp.zeros(
            (_D, _MAXLEN + 128), jnp.float32)

    @pl.when(i < _NBLK)
    def _mlp():
        fs = feats_ref[...]
        h = jnp.maximum(
            jnp.dot(W1t_ref[...], fs, preferred_element_type=jnp.float32)
            + b1_ref[...], 0.0)
        z = (jnp.dot(W2t_ref[...], h, preferred_element_type=jnp.float32)
             + b2_ref[...])
        z_scr[:, pl.ds(i * _BLK, _BLK)] = z

    @pl.when(i >= _NBLK)
    def _cov():
        for bb in range(_CB):
            b = (i - _NBLK) * _CB + bb
            phase = offs_ref[1, b]
            cnt = cnts_ref[0, b]
            zw = z_scr[:, pl.ds(offs_ref[0, b] * 128, _MAXLEN + 128)]
            n = jax.lax.broadcasted_iota(jnp.int32, (1, _MAXLEN + 128), 1)
            maskf = ((n >= phase) & (n < phase + cnt)).astype(jnp.float32)
            zm = zw * maskf
            lfeat_ref[bb] = pltpu.roll(
                zm, (_MAXLEN + 128) - phase, axis=1)[:, :_MAXLEN]
            cf = cnt.astype(jnp.float32)
            mean = jnp.sum(zm, axis=1, keepdims=True) / jnp.maximum(cf, 1.0)
            c = (zw - mean) * maskf
            cov = jax.lax.dot_general(
                c, c, (((1,), (1,)), ((), ())),
                preferred_element_type=jnp.float32) / jnp.maximum(cf - 1.0, 1.0)
            v = jnp.sign(cov) * jnp.sqrt(jnp.abs(cov) + 1e-12)
            out_ref[bb] = v / jnp.maximum(jnp.sqrt(jnp.sum(v * v)), 1e-12)


def kernel(feats, batch_idx, W1, b1, W2, b2):
    bidx2d = batch_idx.reshape(_T // 128, 128)
    out, lfeat = pl.pallas_call(
        _spcov_body,
        grid=(_NBLK + _B // _CB,),
        in_specs=[
            pl.BlockSpec((_T // 128, 128), lambda i: (0, 0)),
            pl.BlockSpec((_IN, _BLK), lambda i: (0, jnp.minimum(i, _NBLK - 1))),
            pl.BlockSpec((_HID, _IN), lambda i: (0, 0)),
            pl.BlockSpec((_HID, 1), lambda i: (0, 0)),
            pl.BlockSpec((_D, _HID), lambda i: (0, 0)),
            pl.BlockSpec((_D, 1), lambda i: (0, 0)),
        ],
        out_specs=[
            pl.BlockSpec((_CB, _D, _D),
                         lambda i: (jnp.maximum(i - _NBLK, 0), 0, 0)),
            pl.BlockSpec((_CB, _D, _MAXLEN),
                         lambda i: (jnp.maximum(i - _NBLK, 0), 0, 0)),
        ],
        out_shape=[
            jax.ShapeDtypeStruct((_B, _D, _D), jnp.float32),
            jax.ShapeDtypeStruct((_B, _D, _MAXLEN), jnp.float32),
        ],
        scratch_shapes=[
            pltpu.VMEM((_D, _T + _MAXLEN + 128), jnp.float32),
            pltpu.SMEM((2, _B), jnp.int32),
            pltpu.SMEM((1, _B), jnp.int32),
        ],
        compiler_params=pltpu.CompilerParams(
            dimension_semantics=("arbitrary",)),
    )(bidx2d, feats.T, W1.T, b1.reshape(_HID, 1), W2.T, b2.reshape(_D, 1))
    return out.reshape(_B, _D * _D), lfeat.transpose(0, 2, 1)
